# baseline (speedup 1.0000x reference)
import functools

import jax
import jax.numpy as jnp
from jax import lax
from jax.experimental import pallas as pl
from jax.experimental.pallas import tpu as pltpu
from jax.experimental.pallas import tpu_sc as plsc

_EPS = 4e-3
_MIN_NORM = 1e-15
_CH = 112
_TPAD = 224


def _logmap_scale(x):
    ss = jnp.sum(x * x, axis=-1, keepdims=True)
    norm = jnp.maximum(jnp.sqrt(ss), _MIN_NORM)
    arg = jnp.minimum(norm, 1.0 - 1e-7)
    return 0.5 * jnp.log((1.0 + arg) / (1.0 - arg)) / norm


def _pack32(t):
    a = jax.lax.bitcast_convert_type(
        t[:, 0:32].astype(jnp.bfloat16), jnp.uint16
    ).astype(jnp.uint32)
    b = jax.lax.bitcast_convert_type(
        t[:, 32:64].astype(jnp.bfloat16), jnp.uint16
    ).astype(jnp.uint32)
    return (a | (b << 16)).astype(jnp.int32)


def _logmap_table_body(x0_ref, x1_ref, x2_ref, x3_ref, out_ref):
    for q, r in enumerate((x0_ref, x1_ref, x2_ref, x3_ref)):
        x = r[...]
        out_ref[:, q * 32:(q + 1) * 32] = _pack32(x * _logmap_scale(x))


def _logmap_table(emb):
    v, d = emb.shape
    vq = v // 4
    blk = 2000
    nblk = vq // blk
    assert vq % blk == 0 and d == 64
    in_specs = [
        pl.BlockSpec((blk, d), functools.partial(lambda q, i: (q * nblk + i, 0), q))
        for q in range(4)
    ]
    return pl.pallas_call(
        _logmap_table_body,
        grid=(nblk,),
        in_specs=in_specs,
        out_specs=pl.BlockSpec((blk, 2 * d), lambda i: (i, 0)),
        out_shape=jax.ShapeDtypeStruct((vq, 2 * d), jnp.int32),
    )(emb, emb, emb, emb)


def _finalize_body(t_tokens, npad, sum_ref, t0_ref, out_ref):
    w = t0_ref[...]
    lo = jax.lax.bitcast_convert_type(w << 16, jnp.float32)
    hi = jax.lax.bitcast_convert_type(w & jnp.int32(-65536), jnp.float32)
    t0 = jnp.concatenate([lo, hi], axis=1)
    u = (sum_ref[...] - npad * t0) * (1.0 / t_tokens)
    ss = jnp.sum(u * u, axis=-1, keepdims=True)
    norm = jnp.maximum(jnp.sqrt(ss), _MIN_NORM)
    y = jnp.tanh(norm) * u / norm
    ssy = jnp.sum(y * y, axis=-1, keepdims=True)
    ny = jnp.maximum(jnp.sqrt(ssy), _MIN_NORM)
    maxn = 1.0 - _EPS
    out_ref[...] = jnp.where(ny > maxn, y / ny * maxn, y)


def _finalize(sums, t0pack, t_tokens, npad):
    b, d = sums.shape
    return pl.pallas_call(
        functools.partial(_finalize_body, float(t_tokens), float(npad)),
        out_shape=jax.ShapeDtypeStruct((b, d), jnp.float32),
    )(sums, t0pack)


def _sc_gather_sum(tang4, ids2, off2, b, d):
    info = plsc.get_sparse_core_info()
    nw = info.num_cores * info.num_subcores
    assert b % nw == 0 and d == 64
    sper = b // nw
    cps = _TPAD // _CH
    cpw = sper * cps
    ngrp = _CH // 16
    assert cps == 2 and _CH % 16 == 0

    mesh = plsc.VectorSubcoreMesh(core_axis_name="c", subcore_axis_name="s")

    @functools.partial(
        pl.kernel,
        out_type=jax.ShapeDtypeStruct((b, d), jnp.float32),
        mesh=mesh,
        scratch_types=[
            pltpu.VMEM((cpw, _CH), jnp.int32),
            pltpu.VMEM((cpw, _CH), jnp.int32),
            pltpu.VMEM((2, _CH, 128), jnp.int32),
            pltpu.VMEM((sper, d), jnp.float32),
            pltpu.SemaphoreType.DMA,
            pltpu.SemaphoreType.DMA,
        ],
    )
    def k(tang_hbm, idx_hbm, off_hbm, out_hbm,
          idx_v, off_v, rows_v, out_v, sem0, sem1):
        wid = lax.axis_index("s") * info.num_cores + lax.axis_index("c")
        base_chunk = wid * cpw
        pltpu.sync_copy(idx_hbm.at[pl.ds(base_chunk, cpw)], idx_v)
        pltpu.sync_copy(off_hbm.at[pl.ds(base_chunk, cpw)], off_v)
        sems = (sem0, sem1)
        for bslot in range(2):
            pltpu.async_copy(
                tang_hbm.at[idx_v.at[bslot]], rows_v.at[bslot], sems[bslot]
            )

        def sentence(s, _):
            zero = jnp.zeros((16,), jnp.float32)
            acc = (zero, zero, zero, zero)
            for bslot in range(2):
                c = 2 * s + bslot
                pltpu.make_async_copy(
                    tang_hbm.at[idx_v.at[c]], rows_v.at[bslot], sems[bslot]
                ).wait()

                def group(g, a, bslot=bslot, c=c):
                    offvec = off_v[c, pl.ds(g * 16, 16)]
                    new = list(a)
                    for j in range(16):
                        off = offvec[j]
                        tok = g * 16 + j
                        for m in range(2):
                            x = rows_v[bslot, tok, pl.ds(off + m * 16, 16)]
                            lo = jax.lax.bitcast_convert_type(
                                x << 16, jnp.float32
                            )
                            hi = jax.lax.bitcast_convert_type(
                                x & jnp.int32(-65536), jnp.float32
                            )
                            new[m] = new[m] + lo
                            new[2 + m] = new[2 + m] + hi
                    return tuple(new)

                acc = lax.fori_loop(0, ngrp, group, acc)

                @pl.when(s < sper - 1)
                def _(bslot=bslot, c=c):
                    pltpu.async_copy(
                        tang_hbm.at[idx_v.at[c + 2]],
                        rows_v.at[bslot],
                        sems[bslot],
                    )
            for k4 in range(4):
                out_v[s, pl.ds(k4 * 16, 16)] = acc[k4]
            return 0

        lax.fori_loop(0, sper, sentence, 0)
        pltpu.sync_copy(out_v, out_hbm.at[pl.ds(wid * sper, sper)])

    return k(tang4, ids2, off2)


def kernel(input_ids, emb):
    b, t_tokens = input_ids.shape
    v, d = emb.shape
    vq = v // 4
    npad = _TPAD - t_tokens
    assert npad >= 0
    tang4 = _logmap_table(emb)
    res = input_ids // vq
    rows = input_ids - res * vq
    offs = res * 32
    pad = jnp.zeros((b, npad), jnp.int32)
    rows = jnp.concatenate([rows, pad], axis=1).reshape(b * _TPAD // _CH, _CH)
    offs = jnp.concatenate([offs, pad], axis=1).reshape(b * _TPAD // _CH, _CH)
    sums = _sc_gather_sum(tang4, rows, offs, b, d)
    t0pack = jax.lax.slice(tang4, (0, 0), (1, 32))
    return _finalize(sums, t0pack, t_tokens, npad)


# speedup vs baseline: 3.8804x; 3.8804x over previous
import functools

import jax
import jax.numpy as jnp
from jax import lax
from jax.experimental import pallas as pl
from jax.experimental.pallas import tpu as pltpu
from jax.experimental.pallas import tpu_sc as plsc

_EPS = 4e-3
_MIN_NORM = 1e-15
_CH = 100


def _logmap_scale(x):
    ss = jnp.sum(x * x, axis=-1, keepdims=True)
    norm = jnp.maximum(jnp.sqrt(ss), _MIN_NORM)
    arg = jnp.minimum(norm, 1.0 - 1e-7)
    return 0.5 * jnp.log((1.0 + arg) / (1.0 - arg)) / norm


def _pack32(t):
    a = jax.lax.bitcast_convert_type(
        t[:, 0:32].astype(jnp.bfloat16), jnp.uint16
    ).astype(jnp.uint32)
    b = jax.lax.bitcast_convert_type(
        t[:, 32:64].astype(jnp.bfloat16), jnp.uint16
    ).astype(jnp.uint32)
    return (a | (b << 16)).astype(jnp.int32)


def _logmap_table_body(x0_ref, x1_ref, x2_ref, x3_ref, out_ref):
    for q, r in enumerate((x0_ref, x1_ref, x2_ref, x3_ref)):
        x = r[...]
        out_ref[:, q * 32:(q + 1) * 32] = _pack32(x * _logmap_scale(x))


def _logmap_table(emb):
    v, d = emb.shape
    vq = v // 4
    blk = 2000
    nblk = vq // blk
    assert vq % blk == 0 and d == 64
    in_specs = [
        pl.BlockSpec((blk, d), functools.partial(lambda q, i: (q * nblk + i, 0), q))
        for q in range(4)
    ]
    return pl.pallas_call(
        _logmap_table_body,
        grid=(nblk,),
        in_specs=in_specs,
        out_specs=pl.BlockSpec((blk, 2 * d), lambda i: (i, 0)),
        out_shape=jax.ShapeDtypeStruct((vq, 2 * d), jnp.int32),
    )(emb, emb, emb, emb)


def _finalize_body(t_tokens, sum_ref, out_ref):
    u = sum_ref[...] * (1.0 / t_tokens)
    ss = jnp.sum(u * u, axis=-1, keepdims=True)
    norm = jnp.maximum(jnp.sqrt(ss), _MIN_NORM)
    y = jnp.tanh(norm) * u / norm
    ssy = jnp.sum(y * y, axis=-1, keepdims=True)
    ny = jnp.maximum(jnp.sqrt(ssy), _MIN_NORM)
    maxn = 1.0 - _EPS
    out_ref[...] = jnp.where(ny > maxn, y / ny * maxn, y)


def _finalize(sums, t_tokens):
    b, d = sums.shape
    return pl.pallas_call(
        functools.partial(_finalize_body, float(t_tokens)),
        out_shape=jax.ShapeDtypeStruct((b, d), jnp.float32),
    )(sums)


def _sc_gather_sum(tang4, ids2, b1, b2, b3, b, t_tokens, d):
    info = plsc.get_sparse_core_info()
    nw = info.num_cores * info.num_subcores
    assert b % nw == 0 and t_tokens == 2 * _CH and d == 64
    sper = b // nw
    cpw = sper * 2
    ngrp = sper // 16
    assert sper % 16 == 0

    mesh = plsc.VectorSubcoreMesh(core_axis_name="c", subcore_axis_name="s")

    @functools.partial(
        pl.kernel,
        out_type=jax.ShapeDtypeStruct((b, d), jnp.float32),
        mesh=mesh,
        scratch_types=[
            pltpu.VMEM((cpw, _CH), jnp.int32),
            pltpu.VMEM((sper,), jnp.int32),
            pltpu.VMEM((sper,), jnp.int32),
            pltpu.VMEM((sper,), jnp.int32),
            pltpu.VMEM((2, _CH, 128), jnp.int32),
            pltpu.VMEM((sper, d), jnp.float32),
            pltpu.SemaphoreType.DMA,
            pltpu.SemaphoreType.DMA,
        ],
    )
    def k(tang_hbm, idx_hbm, b1_hbm, b2_hbm, b3_hbm, out_hbm,
          idx_v, b1_v, b2_v, b3_v, rows_v, out_v, sem0, sem1):
        wid = lax.axis_index("s") * info.num_cores + lax.axis_index("c")
        base_chunk = wid * cpw
        pltpu.sync_copy(idx_hbm.at[pl.ds(base_chunk, cpw)], idx_v)
        pltpu.sync_copy(b1_hbm.at[pl.ds(wid * sper, sper)], b1_v)
        pltpu.sync_copy(b2_hbm.at[pl.ds(wid * sper, sper)], b2_v)
        pltpu.sync_copy(b3_hbm.at[pl.ds(wid * sper, sper)], b3_v)
        sems = (sem0, sem1)
        for bslot in range(2):
            pltpu.async_copy(
                tang_hbm.at[idx_v.at[bslot]], rows_v.at[bslot], sems[bslot]
            )

        def class_loop(bslot, lo, hi, off, acc):
            def body(tok, a):
                new = list(a)
                for m in range(2):
                    x = rows_v[bslot, tok, pl.ds(off + m * 16, 16)]
                    lo_f = jax.lax.bitcast_convert_type(x << 16, jnp.float32)
                    hi_f = jax.lax.bitcast_convert_type(
                        x & jnp.int32(-65536), jnp.float32
                    )
                    new[m] = new[m] + lo_f
                    new[2 + m] = new[2 + m] + hi_f
                return tuple(new)

            return lax.fori_loop(lo, hi, body, acc)

        def group(g, _):
            v1 = b1_v[pl.ds(g * 16, 16)]
            v2 = b2_v[pl.ds(g * 16, 16)]
            v3 = b3_v[pl.ds(g * 16, 16)]
            for j in range(16):
                s = g * 16 + j
                bnd = (0, v1[j], v2[j], v3[j], t_tokens)
                zero = jnp.zeros((16,), jnp.float32)
                acc = (zero, zero, zero, zero)
                for bslot in range(2):
                    c = 2 * s + bslot
                    pltpu.make_async_copy(
                        tang_hbm.at[idx_v.at[c]], rows_v.at[bslot], sems[bslot]
                    ).wait()
                    base = bslot * _CH
                    for q in range(4):
                        lo = jnp.clip(bnd[q] - base, 0, _CH)
                        hi = jnp.clip(bnd[q + 1] - base, 0, _CH)
                        acc = class_loop(bslot, lo, hi, q * 32, acc)

                    @pl.when(s < sper - 1)
                    def _(bslot=bslot, c=c):
                        pltpu.async_copy(
                            tang_hbm.at[idx_v.at[c + 2]],
                            rows_v.at[bslot],
                            sems[bslot],
                        )
                for k4 in range(4):
                    out_v[s, pl.ds(k4 * 16, 16)] = acc[k4]
            return 0

        lax.fori_loop(0, ngrp, group, 0)
        pltpu.sync_copy(out_v, out_hbm.at[pl.ds(wid * sper, sper)])

    return k(tang4, ids2, b1, b2, b3)


def kernel(input_ids, emb):
    b, t_tokens = input_ids.shape
    v, d = emb.shape
    vq = v // 4
    tang4 = _logmap_table(emb)
    srt = jnp.sort(input_ids, axis=1)
    res = srt // vq
    rows = (srt - res * vq).reshape(b * t_tokens // _CH, _CH)
    b1 = jnp.sum((input_ids < vq).astype(jnp.int32), axis=1)
    b2 = jnp.sum((input_ids < 2 * vq).astype(jnp.int32), axis=1)
    b3 = jnp.sum((input_ids < 3 * vq).astype(jnp.int32), axis=1)
    sums = _sc_gather_sum(tang4, rows, b1, b2, b3, b, t_tokens, d)
    return _finalize(sums, t_tokens)


# speedup vs baseline: 4.0220x; 1.0365x over previous
import functools

import jax
import jax.numpy as jnp
from jax import lax
from jax.experimental import pallas as pl
from jax.experimental.pallas import tpu as pltpu
from jax.experimental.pallas import tpu_sc as plsc

_EPS = 4e-3
_MIN_NORM = 1e-15
_CH = 100


def _logmap_scale(x):
    ss = jnp.sum(x * x, axis=-1, keepdims=True)
    norm = jnp.maximum(jnp.sqrt(ss), _MIN_NORM)
    arg = jnp.minimum(norm, 1.0 - 1e-7)
    return 0.5 * jnp.log((1.0 + arg) / (1.0 - arg)) / norm


def _pack32(t):
    bi = jax.lax.bitcast_convert_type(t, jnp.int32)
    a = bi[:, 0:32] + jnp.int32(0x8000)
    b = bi[:, 32:64] + jnp.int32(0x8000)
    lo = jax.lax.shift_right_logical(a, jnp.int32(16))
    hi = b & jnp.int32(-65536)
    return lo | hi


def _logmap_table_body(x0_ref, x1_ref, x2_ref, x3_ref, out_ref):
    for q, r in enumerate((x0_ref, x1_ref, x2_ref, x3_ref)):
        x = r[...]
        out_ref[:, q * 32:(q + 1) * 32] = _pack32(x * _logmap_scale(x))


def _logmap_table(emb):
    v, d = emb.shape
    vq = v // 4
    blk = 5000
    nblk = vq // blk
    assert vq % blk == 0 and d == 64
    in_specs = [
        pl.BlockSpec((blk, d), functools.partial(lambda q, i: (q * nblk + i, 0), q))
        for q in range(4)
    ]
    return pl.pallas_call(
        _logmap_table_body,
        grid=(nblk,),
        in_specs=in_specs,
        out_specs=pl.BlockSpec((blk, 2 * d), lambda i: (i, 0)),
        out_shape=jax.ShapeDtypeStruct((vq, 2 * d), jnp.int32),
    )(emb, emb, emb, emb)


def _finalize_body(t_tokens, sum_ref, out_ref):
    u = sum_ref[...] * (1.0 / t_tokens)
    ss = jnp.sum(u * u, axis=-1, keepdims=True)
    norm = jnp.maximum(jnp.sqrt(ss), _MIN_NORM)
    y = jnp.tanh(norm) * u / norm
    ssy = jnp.sum(y * y, axis=-1, keepdims=True)
    ny = jnp.maximum(jnp.sqrt(ssy), _MIN_NORM)
    maxn = 1.0 - _EPS
    out_ref[...] = jnp.where(ny > maxn, y / ny * maxn, y)


def _finalize(sums, t_tokens):
    b, d = sums.shape
    return pl.pallas_call(
        functools.partial(_finalize_body, float(t_tokens)),
        out_shape=jax.ShapeDtypeStruct((b, d), jnp.float32),
    )(sums)


def _sc_gather_sum(tang4, ids2, b1, b2, b3, b, t_tokens, d):
    info = plsc.get_sparse_core_info()
    nw = info.num_cores * info.num_subcores
    assert b % nw == 0 and t_tokens == 2 * _CH and d == 64
    sper = b // nw
    cpw = sper * 2
    ngrp = sper // 16
    assert sper % 16 == 0

    mesh = plsc.VectorSubcoreMesh(core_axis_name="c", subcore_axis_name="s")

    @functools.partial(
        pl.kernel,
        out_type=jax.ShapeDtypeStruct((b, d), jnp.float32),
        mesh=mesh,
        scratch_types=[
            pltpu.VMEM((cpw, _CH), jnp.int32),
            pltpu.VMEM((sper,), jnp.int32),
            pltpu.VMEM((sper,), jnp.int32),
            pltpu.VMEM((sper,), jnp.int32),
            pltpu.VMEM((2, _CH, 128), jnp.int32),
            pltpu.VMEM((sper, d), jnp.float32),
            pltpu.SemaphoreType.DMA,
            pltpu.SemaphoreType.DMA,
        ],
    )
    def k(tang_hbm, idx_hbm, b1_hbm, b2_hbm, b3_hbm, out_hbm,
          idx_v, b1_v, b2_v, b3_v, rows_v, out_v, sem0, sem1):
        wid = lax.axis_index("s") * info.num_cores + lax.axis_index("c")
        base_chunk = wid * cpw
        pltpu.sync_copy(idx_hbm.at[pl.ds(base_chunk, cpw)], idx_v)
        pltpu.sync_copy(b1_hbm.at[pl.ds(wid * sper, sper)], b1_v)
        pltpu.sync_copy(b2_hbm.at[pl.ds(wid * sper, sper)], b2_v)
        pltpu.sync_copy(b3_hbm.at[pl.ds(wid * sper, sper)], b3_v)
        sems = (sem0, sem1)
        for bslot in range(2):
            pltpu.async_copy(
                tang_hbm.at[idx_v.at[bslot]], rows_v.at[bslot], sems[bslot]
            )

        def class_loop(bslot, lo, hi, off, acc):
            def body(tok, a):
                new = list(a)
                for m in range(2):
                    x = rows_v[bslot, tok, pl.ds(off + m * 16, 16)]
                    lo_f = jax.lax.bitcast_convert_type(x << 16, jnp.float32)
                    hi_f = jax.lax.bitcast_convert_type(
                        x & jnp.int32(-65536), jnp.float32
                    )
                    new[m] = new[m] + lo_f
                    new[2 + m] = new[2 + m] + hi_f
                return tuple(new)

            return lax.fori_loop(lo, hi, body, acc)

        def group(g, _):
            v1 = b1_v[pl.ds(g * 16, 16)]
            v2 = b2_v[pl.ds(g * 16, 16)]
            v3 = b3_v[pl.ds(g * 16, 16)]
            for j in range(16):
                s = g * 16 + j
                bnd = (0, v1[j], v2[j], v3[j], t_tokens)
                zero = jnp.zeros((16,), jnp.float32)
                acc = (zero, zero, zero, zero)
                for bslot in range(2):
                    c = 2 * s + bslot
                    pltpu.make_async_copy(
                        tang_hbm.at[idx_v.at[c]], rows_v.at[bslot], sems[bslot]
                    ).wait()
                    base = bslot * _CH
                    for q in range(4):
                        lo = jnp.clip(bnd[q] - base, 0, _CH)
                        hi = jnp.clip(bnd[q + 1] - base, 0, _CH)
                        acc = class_loop(bslot, lo, hi, q * 32, acc)

                    @pl.when(s < sper - 1)
                    def _(bslot=bslot, c=c):
                        pltpu.async_copy(
                            tang_hbm.at[idx_v.at[c + 2]],
                            rows_v.at[bslot],
                            sems[bslot],
                        )
                for k4 in range(4):
                    out_v[s, pl.ds(k4 * 16, 16)] = acc[k4]
            return 0

        lax.fori_loop(0, ngrp, group, 0)
        pltpu.sync_copy(out_v, out_hbm.at[pl.ds(wid * sper, sper)])

    return k(tang4, ids2, b1, b2, b3)


def kernel(input_ids, emb):
    b, t_tokens = input_ids.shape
    v, d = emb.shape
    vq = v // 4
    tang4 = _logmap_table(emb)
    srt = jnp.sort(input_ids, axis=1)
    res = srt // vq
    rows = (srt - res * vq).reshape(b * t_tokens // _CH, _CH)
    b1 = jnp.sum((input_ids < vq).astype(jnp.int32), axis=1)
    b2 = jnp.sum((input_ids < 2 * vq).astype(jnp.int32), axis=1)
    b3 = jnp.sum((input_ids < 3 * vq).astype(jnp.int32), axis=1)
    sums = _sc_gather_sum(tang4, rows, b1, b2, b3, b, t_tokens, d)
    return _finalize(sums, t_tokens)


# speedup vs baseline: 4.8475x; 1.2052x over previous
import functools

import jax
import jax.numpy as jnp
from jax import lax
from jax.experimental import pallas as pl
from jax.experimental.pallas import tpu as pltpu
from jax.experimental.pallas import tpu_sc as plsc

_EPS = 4e-3
_MIN_NORM = 1e-15
_CH = 100


def _logmap_table_body(x_ref, out_ref):
    x = x_ref[...]
    ss = jnp.sum(x * x, axis=-1, keepdims=True)
    norm = jnp.maximum(jnp.sqrt(ss), _MIN_NORM)
    arg = jnp.minimum(norm, 1.0 - 1e-7)
    t = x * (0.5 * jnp.log((1.0 + arg) / (1.0 - arg)) / norm)
    out_ref[:, 0:64] = t
    out_ref[:, 64:128] = t


def _logmap_table(emb):
    v, d = emb.shape
    blk = 8000
    nblk = v // blk
    assert v % blk == 0 and d == 64
    return pl.pallas_call(
        _logmap_table_body,
        grid=(nblk,),
        in_specs=[pl.BlockSpec((blk, d), lambda i: (i, 0))],
        out_specs=pl.BlockSpec((blk, 2 * d), lambda i: (i, 0)),
        out_shape=jax.ShapeDtypeStruct((v, 2 * d), jnp.float32),
    )(emb)


def _finalize_body(t_tokens, sum_ref, out_ref):
    u = sum_ref[...] * (1.0 / t_tokens)
    ss = jnp.sum(u * u, axis=-1, keepdims=True)
    norm = jnp.maximum(jnp.sqrt(ss), _MIN_NORM)
    y = jnp.tanh(norm) * u / norm
    ssy = jnp.sum(y * y, axis=-1, keepdims=True)
    ny = jnp.maximum(jnp.sqrt(ssy), _MIN_NORM)
    maxn = 1.0 - _EPS
    out_ref[...] = jnp.where(ny > maxn, y / ny * maxn, y)


def _finalize(sums, t_tokens):
    b, d = sums.shape
    return pl.pallas_call(
        functools.partial(_finalize_body, float(t_tokens)),
        out_shape=jax.ShapeDtypeStruct((b, d), jnp.float32),
    )(sums)


def _sc_gather_sum(tang2, ids2, b, t_tokens, d):
    info = plsc.get_sparse_core_info()
    nw = info.num_cores * info.num_subcores
    assert b % nw == 0 and t_tokens % _CH == 0 and d == 64
    sper = b // nw
    cps = t_tokens // _CH
    cpw = sper * cps
    assert cps == 2

    mesh = plsc.VectorSubcoreMesh(core_axis_name="c", subcore_axis_name="s")

    @functools.partial(
        pl.kernel,
        out_type=jax.ShapeDtypeStruct((b, d), jnp.float32),
        mesh=mesh,
        scratch_types=[
            pltpu.VMEM((cpw, _CH), jnp.int32),
            pltpu.VMEM((2, _CH, 2 * d), jnp.float32),
            pltpu.VMEM((sper, d), jnp.float32),
            pltpu.SemaphoreType.DMA,
            pltpu.SemaphoreType.DMA,
        ],
    )
    def k(tang_hbm, idx_hbm, out_hbm, idx_v, rows_v, out_v, sem0, sem1):
        wid = lax.axis_index("s") * info.num_cores + lax.axis_index("c")
        base_chunk = wid * cpw
        pltpu.sync_copy(idx_hbm.at[pl.ds(base_chunk, cpw)], idx_v)
        sems = (sem0, sem1)
        for bslot in range(2):
            pltpu.async_copy(
                tang_hbm.at[idx_v.at[bslot]], rows_v.at[bslot], sems[bslot]
            )

        def sentence(s, _):
            zero = jnp.zeros((16,), jnp.float32)
            acc = (zero, zero, zero, zero)
            for bslot in range(2):
                c = 2 * s + bslot
                pltpu.make_async_copy(
                    tang_hbm.at[idx_v.at[c]], rows_v.at[bslot], sems[bslot]
                ).wait()

                def body(tok, a, bslot=bslot):
                    return tuple(
                        a[k] + rows_v[bslot, tok, pl.ds(k * 16, 16)]
                        for k in range(4)
                    )

                acc = lax.fori_loop(0, _CH, body, acc, unroll=4)

                @pl.when(s < sper - 1)
                def _(bslot=bslot, c=c):
                    pltpu.async_copy(
                        tang_hbm.at[idx_v.at[c + 2]],
                        rows_v.at[bslot],
                        sems[bslot],
                    )
            for k4 in range(4):
                out_v[s, pl.ds(k4 * 16, 16)] = acc[k4]
            return 0

        lax.fori_loop(0, sper, sentence, 0)
        pltpu.sync_copy(out_v, out_hbm.at[pl.ds(wid * sper, sper)])

    return k(tang2, ids2)


def kernel(input_ids, emb):
    b, t_tokens = input_ids.shape
    v, d = emb.shape
    tang2 = _logmap_table(emb)
    ids2 = input_ids.reshape(b * t_tokens // _CH, _CH)
    sums = _sc_gather_sum(tang2, ids2, b, t_tokens, d)
    return _finalize(sums, t_tokens)


# speedup vs baseline: 4.9171x; 1.0144x over previous
import functools

import jax
import jax.numpy as jnp
from jax import lax
from jax.experimental import pallas as pl
from jax.experimental.pallas import tpu as pltpu
from jax.experimental.pallas import tpu_sc as plsc

_EPS = 4e-3
_MIN_NORM = 1e-15
_CH = 100


def _logmap_table_body(x_ref, out_ref):
    x = x_ref[...]
    ss = jnp.sum(x * x, axis=-1, keepdims=True)
    norm = jnp.maximum(jnp.sqrt(ss), _MIN_NORM)
    arg = jnp.minimum(norm, 1.0 - 1e-7)
    t = x * (0.5 * jnp.log((1.0 + arg) / (1.0 - arg)) / norm)
    out_ref[:, 0:64] = t
    out_ref[:, 64:128] = t


def _logmap_table(emb):
    v, d = emb.shape
    blk = 10000
    nblk = v // blk
    assert v % blk == 0 and d == 64
    return pl.pallas_call(
        _logmap_table_body,
        grid=(nblk,),
        in_specs=[pl.BlockSpec((blk, d), lambda i: (i, 0))],
        out_specs=pl.BlockSpec((blk, 2 * d), lambda i: (i, 0)),
        out_shape=jax.ShapeDtypeStruct((v, 2 * d), jnp.float32),
    )(emb)


def _finalize_body(t_tokens, sum_ref, out_ref):
    u = sum_ref[...] * (1.0 / t_tokens)
    ss = jnp.sum(u * u, axis=-1, keepdims=True)
    norm = jnp.maximum(jnp.sqrt(ss), _MIN_NORM)
    y = jnp.tanh(norm) * u / norm
    ssy = jnp.sum(y * y, axis=-1, keepdims=True)
    ny = jnp.maximum(jnp.sqrt(ssy), _MIN_NORM)
    maxn = 1.0 - _EPS
    out_ref[...] = jnp.where(ny > maxn, y / ny * maxn, y)


def _finalize(sums, t_tokens):
    b, d = sums.shape
    return pl.pallas_call(
        functools.partial(_finalize_body, float(t_tokens)),
        out_shape=jax.ShapeDtypeStruct((b, d), jnp.float32),
    )(sums)


def _sc_gather_sum(tang2, ids2, b, t_tokens, d):
    info = plsc.get_sparse_core_info()
    nw = info.num_cores * info.num_subcores
    assert b % nw == 0 and t_tokens % _CH == 0 and d == 64
    sper = b // nw
    cps = t_tokens // _CH
    cpw = sper * cps
    assert cps == 2

    mesh = plsc.VectorSubcoreMesh(core_axis_name="c", subcore_axis_name="s")

    @functools.partial(
        pl.kernel,
        out_type=jax.ShapeDtypeStruct((b, d), jnp.float32),
        mesh=mesh,
        scratch_types=[
            pltpu.VMEM((cpw, _CH), jnp.int32),
            pltpu.VMEM((2, _CH, 2 * d), jnp.float32),
            pltpu.VMEM((sper, d), jnp.float32),
            pltpu.SemaphoreType.DMA,
            pltpu.SemaphoreType.DMA,
        ],
    )
    def k(tang_hbm, idx_hbm, out_hbm, idx_v, rows_v, out_v, sem0, sem1):
        wid = lax.axis_index("s") * info.num_cores + lax.axis_index("c")
        base_chunk = wid * cpw
        pltpu.sync_copy(idx_hbm.at[pl.ds(base_chunk, cpw)], idx_v)
        sems = (sem0, sem1)
        for bslot in range(2):
            pltpu.async_copy(
                tang_hbm.at[idx_v.at[bslot]], rows_v.at[bslot], sems[bslot]
            )

        def sentence(s, _):
            zero = jnp.zeros((16,), jnp.float32)
            acc = (zero, zero, zero, zero)
            for bslot in range(2):
                c = 2 * s + bslot
                pltpu.make_async_copy(
                    tang_hbm.at[idx_v.at[c]], rows_v.at[bslot], sems[bslot]
                ).wait()

                def body(tok, a, bslot=bslot):
                    return tuple(
                        a[k] + rows_v[bslot, tok, pl.ds(k * 16, 16)]
                        for k in range(4)
                    )

                acc = lax.fori_loop(0, _CH, body, acc, unroll=4)

                @pl.when(s < sper - 1)
                def _(bslot=bslot, c=c):
                    pltpu.async_copy(
                        tang_hbm.at[idx_v.at[c + 2]],
                        rows_v.at[bslot],
                        sems[bslot],
                    )
            for k4 in range(4):
                out_v[s, pl.ds(k4 * 16, 16)] = acc[k4]
            return 0

        lax.fori_loop(0, sper, sentence, 0)
        pltpu.sync_copy(out_v, out_hbm.at[pl.ds(wid * sper, sper)])

    return k(tang2, ids2)


def kernel(input_ids, emb):
    b, t_tokens = input_ids.shape
    v, d = emb.shape
    tang2 = _logmap_table(emb)
    ids2 = input_ids.reshape(b * t_tokens // _CH, _CH)
    sums = _sc_gather_sum(tang2, ids2, b, t_tokens, d)
    return _finalize(sums, t_tokens)


# speedup vs baseline: 5.0745x; 1.0320x over previous
import functools

import jax
import jax.numpy as jnp
from jax import lax
from jax.experimental import pallas as pl
from jax.experimental.pallas import tpu as pltpu
from jax.experimental.pallas import tpu_sc as plsc

_EPS = 4e-3
_MIN_NORM = 1e-15
_CH = 100


def _logmap_table_body(x_ref, out_ref):
    x = x_ref[...]
    ss = jnp.sum(x * x, axis=-1, keepdims=True)
    norm = jnp.maximum(jnp.sqrt(ss), _MIN_NORM)
    arg = jnp.minimum(norm, 1.0 - 1e-7)
    t = x * (0.5 * jnp.log((1.0 + arg) / (1.0 - arg)) / norm)
    out_ref[:, 0:64] = t
    out_ref[:, 64:128] = t


def _logmap_table(emb):
    v, d = emb.shape
    blk = 20000
    nblk = v // blk
    assert v % blk == 0 and d == 64
    return pl.pallas_call(
        _logmap_table_body,
        grid=(nblk,),
        in_specs=[pl.BlockSpec((blk, d), lambda i: (i, 0))],
        out_specs=pl.BlockSpec((blk, 2 * d), lambda i: (i, 0)),
        out_shape=jax.ShapeDtypeStruct((v, 2 * d), jnp.float32),
    )(emb)


def _finalize_body(t_tokens, sum_ref, out_ref):
    u = sum_ref[...] * (1.0 / t_tokens)
    ss = jnp.sum(u * u, axis=-1, keepdims=True)
    norm = jnp.maximum(jnp.sqrt(ss), _MIN_NORM)
    y = jnp.tanh(norm) * u / norm
    ssy = jnp.sum(y * y, axis=-1, keepdims=True)
    ny = jnp.maximum(jnp.sqrt(ssy), _MIN_NORM)
    maxn = 1.0 - _EPS
    out_ref[...] = jnp.where(ny > maxn, y / ny * maxn, y)


def _finalize(sums, t_tokens):
    b, d = sums.shape
    return pl.pallas_call(
        functools.partial(_finalize_body, float(t_tokens)),
        out_shape=jax.ShapeDtypeStruct((b, d), jnp.float32),
    )(sums)


def _sc_gather_sum(tang2, ids2, b, t_tokens, d):
    info = plsc.get_sparse_core_info()
    nw = info.num_cores * info.num_subcores
    assert b % nw == 0 and t_tokens % _CH == 0 and d == 64
    sper = b // nw
    cps = t_tokens // _CH
    cpw = sper * cps
    assert cps == 2

    mesh = plsc.VectorSubcoreMesh(core_axis_name="c", subcore_axis_name="s")

    @functools.partial(
        pl.kernel,
        out_type=jax.ShapeDtypeStruct((b, d), jnp.float32),
        mesh=mesh,
        scratch_types=[
            pltpu.VMEM((cpw, _CH), jnp.int32),
            pltpu.VMEM((2, _CH, 2 * d), jnp.float32),
            pltpu.VMEM((sper, d), jnp.float32),
            pltpu.SemaphoreType.DMA,
            pltpu.SemaphoreType.DMA,
        ],
    )
    def k(tang_hbm, idx_hbm, out_hbm, idx_v, rows_v, out_v, sem0, sem1):
        wid = lax.axis_index("s") * info.num_cores + lax.axis_index("c")
        base_chunk = wid * cpw
        pltpu.sync_copy(idx_hbm.at[pl.ds(base_chunk, cpw)], idx_v)
        sems = (sem0, sem1)
        for bslot in range(2):
            pltpu.async_copy(
                tang_hbm.at[idx_v.at[bslot]], rows_v.at[bslot], sems[bslot]
            )

        def sentence(s, _):
            zero = jnp.zeros((16,), jnp.float32)
            acc = (zero, zero, zero, zero)
            for bslot in range(2):
                c = 2 * s + bslot
                pltpu.make_async_copy(
                    tang_hbm.at[idx_v.at[c]], rows_v.at[bslot], sems[bslot]
                ).wait()

                def body(tok, a, bslot=bslot):
                    return tuple(
                        a[k] + rows_v[bslot, tok, pl.ds(k * 16, 16)]
                        for k in range(4)
                    )

                acc = lax.fori_loop(0, _CH, body, acc, unroll=4)

                @pl.when(s < sper - 1)
                def _(bslot=bslot, c=c):
                    pltpu.async_copy(
                        tang_hbm.at[idx_v.at[c + 2]],
                        rows_v.at[bslot],
                        sems[bslot],
                    )
            for k4 in range(4):
                out_v[s, pl.ds(k4 * 16, 16)] = acc[k4]
            return 0

        lax.fori_loop(0, sper, sentence, 0)
        pltpu.sync_copy(out_v, out_hbm.at[pl.ds(wid * sper, sper)])

    return k(tang2, ids2)


def kernel(input_ids, emb):
    b, t_tokens = input_ids.shape
    v, d = emb.shape
    tang2 = _logmap_table(emb)
    ids2 = input_ids.reshape(b * t_tokens // _CH, _CH)
    sums = _sc_gather_sum(tang2, ids2, b, t_tokens, d)
    return _finalize(sums, t_tokens)
